# baseline (speedup 1.0000x reference)
import functools

import jax
import jax.numpy as jnp
from jax import lax
from jax.experimental import pallas as pl
from jax.experimental.pallas import tpu as pltpu
from jax.experimental.pallas import tpu_sc as plsc

_B = 1024
_LEN = 200
_D = 64
_NC, _NS = 2, 16
_NW = _NC * _NS
_SPW = _B // _NW
_CH = 8
_NCH = _LEN // _CH
_L16 = 16


def _sc_body(xt, table, pre, out, xblk, idx_v, stg_v, asm_v, cls_v, sep_v,
             gsem, osem):
    c = lax.axis_index("c")
    s = lax.axis_index("s")
    wid = s * _NC + c
    base = wid * _SPW
    tb = wid // 4
    bc0 = (wid % 4) * _SPW

    pltpu.sync_copy(xt.at[:, pl.ds(base, _SPW)], xblk)
    pltpu.sync_copy(pre.at[0], cls_v)
    pltpu.sync_copy(pre.at[1], sep_v)

    lanes = lax.iota(jnp.int32, _L16)
    hi3 = lax.shift_right_logical(lanes, 3)
    lo3 = lanes & 7

    def xpose_x(p, carry):
        r2 = 2 * (p // _CH)
        col = lanes * _CH + p % _CH
        for h in (0, 1):
            v = xblk[p, pl.ds(h * _L16, _L16)]
            plsc.store_scatter(
                idx_v, [jnp.full((_L16,), r2 + h, jnp.int32), col], v)
        return carry

    lax.fori_loop(0, _LEN, xpose_x, 0)

    def gather_descs(u, sl):
        return tuple(
            pltpu.make_async_copy(
                table.at[idx_v.at[2 * u + h]],
                stg_v.at[sl, pl.ds(h * 128, 128)],
                gsem.at[sl])
            for h in (0, 1)
        )

    def out_desc(u, sl):
        return pltpu.make_async_copy(
            asm_v.at[sl],
            out.at[pl.ds(u * _CH, _CH), :, tb, :, pl.ds(bc0, _SPW)],
            osem.at[sl])

    def xpose_unit(sl):
        def body(pp, carry):
            pp_vec = jnp.full((_L16,), pp, jnp.int32)
            for h in (0, 1):
                for qg in range(_L16):
                    q_vec = jnp.full((_L16,), h * _L16 + qg, jnp.int32)
                    row = h * 128 + qg * _CH
                    for d0 in range(_D // _L16):
                        src = stg_v[sl, row + pp, pl.ds(d0 * _L16, _L16)]
                        plsc.store_scatter(
                            asm_v.at[sl],
                            [pp_vec, 2 * d0 + hi3, lo3, q_vec], src)
            return carry

        lax.fori_loop(0, _CH, body, 0)

    def fixup(sl, pp, vec):
        pp_vec = jnp.full((_L16,), pp, jnp.int32)
        for q in range(_SPW):
            q_vec = jnp.full((_L16,), q, jnp.int32)
            for d0 in range(_D // _L16):
                plsc.store_scatter(
                    asm_v.at[sl],
                    [pp_vec, 2 * d0 + hi3, lo3, q_vec],
                    vec[pl.ds(d0 * _L16, _L16)])

    for u in (0, 1):
        for d in gather_descs(u, u % 2):
            d.start()

    def unit_step(u, sl):
        @pl.when(u >= 2)
        def _():
            out_desc(u - 2, sl).wait()

        for d in gather_descs(u, sl):
            d.wait()
        xpose_unit(sl)

        @pl.when(u == 0)
        def _():
            fixup(sl, 0, cls_v)

        @pl.when(u == _NCH - 1)
        def _():
            fixup(sl, _CH - 1, sep_v)

        out_desc(u, sl).start()

        @pl.when(u + 2 < _NCH)
        def _():
            for d in gather_descs(u + 2, sl):
                d.start()

    def pair_step(i, carry):
        for b in (0, 1):
            unit_step(2 * i + b, b)
        return carry

    lax.fori_loop(0, _NCH // 2, pair_step, 0)
    unit_step(_NCH - 1, 0)

    out_desc(_NCH - 2, 1).wait()
    out_desc(_NCH - 1, 0).wait()


_gather_call = functools.partial(
    pl.kernel,
    out_type=jax.ShapeDtypeStruct((_LEN, _D // 8, _B // 128, 8, 128),
                                  jnp.float32),
    mesh=plsc.VectorSubcoreMesh(core_axis_name="c", subcore_axis_name="s"),
    compiler_params=pltpu.CompilerParams(use_tc_tiling_on_sc=False,
                                         needs_layout_passes=False),
    scratch_types=[
        pltpu.VMEM((_LEN, _SPW), jnp.int32),
        pltpu.VMEM((2 * _NCH, 128), jnp.int32),
        pltpu.VMEM((2, 2 * 128, _D), jnp.float32),
        pltpu.VMEM((2, _CH, _D // 8, 8, _SPW), jnp.float32),
        pltpu.VMEM((_D,), jnp.float32),
        pltpu.VMEM((_D,), jnp.float32),
        pltpu.SemaphoreType.DMA((2,)),
        pltpu.SemaphoreType.DMA((2,)),
    ],
)(_sc_body)


@jax.jit
def kernel(x, node_table, pre_table):
    out5 = _gather_call(x.T.astype(jnp.int32), node_table, pre_table)
    return out5.transpose(2, 4, 0, 1, 3).reshape(_B, _LEN, _D)


# speedup vs baseline: 1.5394x; 1.5394x over previous
import functools

import jax
import jax.numpy as jnp
from jax import lax
from jax.experimental import pallas as pl
from jax.experimental.pallas import tpu as pltpu
from jax.experimental.pallas import tpu_sc as plsc

_B = 1024
_LEN = 200
_D = 64
_NC, _NS = 2, 16
_NW = _NC * _NS
_SPW = _B // _NW
_CH = 8
_NCH = _LEN // _CH
_L16 = 16


def _sc_body(xt, table, pre, out, xblk, idx_v, stg_v, asm_v, cls_v, sep_v,
             gsem, osem):
    c = lax.axis_index("c")
    s = lax.axis_index("s")
    wid = s * _NC + c
    base = wid * _SPW
    tb = wid // 4
    bc0 = (wid % 4) * _SPW

    pltpu.sync_copy(xt.at[:, pl.ds(base, _SPW)], xblk)
    pltpu.sync_copy(pre.at[0], cls_v)
    pltpu.sync_copy(pre.at[1], sep_v)

    lanes = lax.iota(jnp.int32, _L16)
    hi3 = lax.shift_right_logical(lanes, 3)
    lo3 = lanes & 7

    def xpose_x(p, carry):
        r2 = 2 * (p // _CH)
        col = lanes * _CH + p % _CH
        for h in (0, 1):
            v = xblk[p, pl.ds(h * _L16, _L16)]
            plsc.store_scatter(
                idx_v, [jnp.full((_L16,), r2 + h, jnp.int32), col], v)
        return carry

    lax.fori_loop(0, _LEN, xpose_x, 0)

    def gather_descs(u, sl):
        return tuple(
            pltpu.make_async_copy(
                table.at[idx_v.at[2 * u + h]],
                stg_v.at[sl, pl.ds(h * 128, 128)],
                gsem.at[sl])
            for h in (0, 1)
        )

    def out_desc(u, sl):
        return pltpu.make_async_copy(
            asm_v.at[sl, :, :, :, pl.ds(0, _SPW)],
            out.at[pl.ds(u * _CH, _CH), :, tb, :, pl.ds(bc0, _SPW)],
            osem.at[sl])

    def xpose_unit(sl):
        def body(pp, carry):
            pp_vec = jnp.full((_L16,), pp, jnp.int32)
            for h in (0, 1):
                for qg in range(_L16):
                    q_vec = jnp.full((_L16,), h * _L16 + qg, jnp.int32)
                    row = h * 128 + qg * _CH
                    for d0 in range(_D // _L16):
                        src = stg_v[sl, row + pp, pl.ds(d0 * _L16, _L16)]
                        plsc.store_scatter(
                            asm_v.at[sl],
                            [pp_vec, 2 * d0 + hi3, lo3, q_vec], src)
            return carry

        lax.fori_loop(0, _CH, body, 0)

    def fixup(sl, pp, vec):
        pp_vec = jnp.full((_L16,), pp, jnp.int32)
        for q in range(_SPW):
            q_vec = jnp.full((_L16,), q, jnp.int32)
            for d0 in range(_D // _L16):
                plsc.store_scatter(
                    asm_v.at[sl],
                    [pp_vec, 2 * d0 + hi3, lo3, q_vec],
                    vec[pl.ds(d0 * _L16, _L16)])

    for u in (0, 1):
        for d in gather_descs(u, u % 2):
            d.start()

    def unit_step(u, sl):
        @pl.when(u >= 2)
        def _():
            out_desc(u - 2, sl).wait()

        for d in gather_descs(u, sl):
            d.wait()
        xpose_unit(sl)

        @pl.when(u == 0)
        def _():
            fixup(sl, 0, cls_v)

        @pl.when(u == _NCH - 1)
        def _():
            fixup(sl, _CH - 1, sep_v)

        out_desc(u, sl).start()

        @pl.when(u + 2 < _NCH)
        def _():
            for d in gather_descs(u + 2, sl):
                d.start()

    def pair_step(i, carry):
        for b in (0, 1):
            unit_step(2 * i + b, b)
        return carry

    lax.fori_loop(0, _NCH // 2, pair_step, 0)
    unit_step(_NCH - 1, 0)

    out_desc(_NCH - 2, 1).wait()
    out_desc(_NCH - 1, 0).wait()


_gather_call = functools.partial(
    pl.kernel,
    out_type=jax.ShapeDtypeStruct((_LEN, _D // 8, _B // 128, 8, 128),
                                  jnp.float32),
    mesh=plsc.VectorSubcoreMesh(core_axis_name="c", subcore_axis_name="s"),
    compiler_params=pltpu.CompilerParams(use_tc_tiling_on_sc=False,
                                         needs_layout_passes=False),
    scratch_types=[
        pltpu.VMEM((_LEN, _SPW), jnp.int32),
        pltpu.VMEM((2 * _NCH, 128), jnp.int32),
        pltpu.VMEM((2, 2 * 128, _D), jnp.float32),
        pltpu.VMEM((2, _CH, _D // 8, 8, _SPW + 1), jnp.float32),
        pltpu.VMEM((_D,), jnp.float32),
        pltpu.VMEM((_D,), jnp.float32),
        pltpu.SemaphoreType.DMA((2,)),
        pltpu.SemaphoreType.DMA((2,)),
    ],
)(_sc_body)


@jax.jit
def kernel(x, node_table, pre_table):
    out5 = _gather_call(x.T.astype(jnp.int32), node_table, pre_table)
    return out5.transpose(2, 4, 0, 1, 3).reshape(_B, _LEN, _D)


# speedup vs baseline: 1.7761x; 1.1538x over previous
import functools

import jax
import jax.numpy as jnp
from jax import lax
from jax.experimental import pallas as pl
from jax.experimental.pallas import tpu as pltpu
from jax.experimental.pallas import tpu_sc as plsc

_B = 1024
_LEN = 200
_D = 64
_NC, _NS = 2, 16
_NW = _NC * _NS
_SPW = _B // _NW
_CH = 8
_NCH = _LEN // _CH
_L16 = 16


def _sc_body(xt, table, pre, out, xblk, idx_v, stg_v, asm_v, cls_v, sep_v,
             gsem, osem):
    c = lax.axis_index("c")
    s = lax.axis_index("s")
    wid = s * _NC + c
    base = wid * _SPW
    tb = wid // 4
    bc0 = (wid % 4) * _SPW

    pltpu.sync_copy(xt.at[:, pl.ds(base, _SPW)], xblk)
    pltpu.sync_copy(pre.at[0], cls_v)
    pltpu.sync_copy(pre.at[1], sep_v)

    lanes = lax.iota(jnp.int32, _L16)
    hi3 = lax.shift_right_logical(lanes, 3)
    lo3 = lanes & 7

    @plsc.parallel_loop(0, _LEN)
    def xpose_x(p):
        r2 = 2 * (p // _CH)
        col = lanes * _CH + p % _CH
        for h in (0, 1):
            v = xblk[p, pl.ds(h * _L16, _L16)]
            plsc.store_scatter(
                idx_v, [jnp.full((_L16,), r2 + h, jnp.int32), col], v)

    def gather_descs(u, sl):
        return tuple(
            pltpu.make_async_copy(
                table.at[idx_v.at[2 * u + h]],
                stg_v.at[sl, pl.ds(h * 128, 128)],
                gsem.at[sl])
            for h in (0, 1)
        )

    def out_desc(u, sl):
        return pltpu.make_async_copy(
            asm_v.at[sl, :, :, :, pl.ds(0, _SPW)],
            out.at[pl.ds(u * _CH, _CH), :, tb, :, pl.ds(bc0, _SPW)],
            osem.at[sl])

    def xpose_unit(sl):
        @plsc.parallel_loop(0, _CH)
        def body(pp):
            pp_vec = jnp.full((_L16,), pp, jnp.int32)
            for h in (0, 1):
                for qg in range(_L16):
                    q_vec = jnp.full((_L16,), h * _L16 + qg, jnp.int32)
                    row = h * 128 + qg * _CH
                    for d0 in range(_D // _L16):
                        src = stg_v[sl, row + pp, pl.ds(d0 * _L16, _L16)]
                        plsc.store_scatter(
                            asm_v.at[sl],
                            [pp_vec, 2 * d0 + hi3, lo3, q_vec], src)

    def fixup(sl, pp, vec):
        pp_vec = jnp.full((_L16,), pp, jnp.int32)
        for q in range(_SPW):
            q_vec = jnp.full((_L16,), q, jnp.int32)
            for d0 in range(_D // _L16):
                plsc.store_scatter(
                    asm_v.at[sl],
                    [pp_vec, 2 * d0 + hi3, lo3, q_vec],
                    vec[pl.ds(d0 * _L16, _L16)])

    for u in (0, 1):
        for d in gather_descs(u, u % 2):
            d.start()

    def unit_step(u, sl):
        @pl.when(u >= 2)
        def _():
            out_desc(u - 2, sl).wait()

        for d in gather_descs(u, sl):
            d.wait()
        xpose_unit(sl)

        @pl.when(u == 0)
        def _():
            fixup(sl, 0, cls_v)

        @pl.when(u == _NCH - 1)
        def _():
            fixup(sl, _CH - 1, sep_v)

        out_desc(u, sl).start()

        @pl.when(u + 2 < _NCH)
        def _():
            for d in gather_descs(u + 2, sl):
                d.start()

    def pair_step(i, carry):
        for b in (0, 1):
            unit_step(2 * i + b, b)
        return carry

    lax.fori_loop(0, _NCH // 2, pair_step, 0)
    unit_step(_NCH - 1, 0)

    out_desc(_NCH - 2, 1).wait()
    out_desc(_NCH - 1, 0).wait()


_gather_call = functools.partial(
    pl.kernel,
    out_type=jax.ShapeDtypeStruct((_LEN, _D // 8, _B // 128, 8, 128),
                                  jnp.float32),
    mesh=plsc.VectorSubcoreMesh(core_axis_name="c", subcore_axis_name="s"),
    compiler_params=pltpu.CompilerParams(use_tc_tiling_on_sc=False,
                                         needs_layout_passes=False),
    scratch_types=[
        pltpu.VMEM((_LEN, _SPW), jnp.int32),
        pltpu.VMEM((2 * _NCH, 128), jnp.int32),
        pltpu.VMEM((2, 2 * 128, _D), jnp.float32),
        pltpu.VMEM((2, _CH, _D // 8, 8, _SPW + 1), jnp.float32),
        pltpu.VMEM((_D,), jnp.float32),
        pltpu.VMEM((_D,), jnp.float32),
        pltpu.SemaphoreType.DMA((2,)),
        pltpu.SemaphoreType.DMA((2,)),
    ],
)(_sc_body)


@jax.jit
def kernel(x, node_table, pre_table):
    out5 = _gather_call(x.T.astype(jnp.int32), node_table, pre_table)
    return out5.transpose(2, 4, 0, 1, 3).reshape(_B, _LEN, _D)


# speedup vs baseline: 1.9460x; 1.0957x over previous
import functools

import jax
import jax.numpy as jnp
from jax import lax
from jax.experimental import pallas as pl
from jax.experimental.pallas import tpu as pltpu
from jax.experimental.pallas import tpu_sc as plsc

_B = 1024
_LEN = 200
_D = 64
_NC, _NS = 2, 16
_NW = _NC * _NS
_SPW = _B // _NW
_CH = 8
_NCH = _LEN // _CH
_L16 = 16


def _sc_body(xt, table, pre, out, xblk, idx_v, stg_v, asm_v, cls_v, sep_v,
             gsem, osem):
    c = lax.axis_index("c")
    s = lax.axis_index("s")
    wid = s * _NC + c
    base = wid * _SPW
    tb = wid // 4
    bc0 = (wid % 4) * _SPW

    pltpu.sync_copy(xt.at[:, pl.ds(base, _SPW)], xblk)
    pltpu.sync_copy(pre.at[0], cls_v)
    pltpu.sync_copy(pre.at[1], sep_v)

    lanes = lax.iota(jnp.int32, _L16)
    hi3 = lax.shift_right_logical(lanes, 3)
    lo3 = lanes & 7

    @plsc.parallel_loop(0, _LEN)
    def xpose_x(p):
        r2 = 2 * (p // _CH)
        col = lanes * _CH + p % _CH
        for h in (0, 1):
            v = xblk[p, pl.ds(h * _L16, _L16)]
            plsc.store_scatter(
                idx_v, [jnp.full((_L16,), r2 + h, jnp.int32), col], v)

    def gather_descs(u, sl):
        return tuple(
            pltpu.make_async_copy(
                table.at[idx_v.at[2 * u + h]],
                stg_v.at[sl, pl.ds(h * 128, 128)],
                gsem.at[sl])
            for h in (0, 1)
        )

    def out_desc(u, sl):
        return pltpu.make_async_copy(
            asm_v.at[sl, :, :, :, pl.ds(0, _SPW)],
            out.at[pl.ds(u * _CH, _CH), :, tb, :, pl.ds(bc0, _SPW)],
            osem.at[sl])

    def xpose_unit(sl):
        @plsc.parallel_loop(0, _CH, unroll=2)
        def body(pp):
            pp_vec = jnp.full((_L16,), pp, jnp.int32)
            for h in (0, 1):
                for qg in range(_L16):
                    q_vec = jnp.full((_L16,), h * _L16 + qg, jnp.int32)
                    row = h * 128 + qg * _CH
                    for d0 in range(_D // _L16):
                        src = stg_v[sl, row + pp, pl.ds(d0 * _L16, _L16)]
                        plsc.store_scatter(
                            asm_v.at[sl],
                            [pp_vec, 2 * d0 + hi3, lo3, q_vec], src)

    def fixup(sl, pp, vec):
        pp_vec = jnp.full((_L16,), pp, jnp.int32)
        for q in range(_SPW):
            q_vec = jnp.full((_L16,), q, jnp.int32)
            for d0 in range(_D // _L16):
                plsc.store_scatter(
                    asm_v.at[sl],
                    [pp_vec, 2 * d0 + hi3, lo3, q_vec],
                    vec[pl.ds(d0 * _L16, _L16)])

    for u in (0, 1):
        for d in gather_descs(u, u % 2):
            d.start()

    def unit_step(u, sl):
        @pl.when(u >= 2)
        def _():
            out_desc(u - 2, sl).wait()

        for d in gather_descs(u, sl):
            d.wait()
        xpose_unit(sl)

        @pl.when(u == 0)
        def _():
            fixup(sl, 0, cls_v)

        @pl.when(u == _NCH - 1)
        def _():
            fixup(sl, _CH - 1, sep_v)

        out_desc(u, sl).start()

        @pl.when(u + 2 < _NCH)
        def _():
            for d in gather_descs(u + 2, sl):
                d.start()

    def pair_step(i, carry):
        for b in (0, 1):
            unit_step(2 * i + b, b)
        return carry

    lax.fori_loop(0, _NCH // 2, pair_step, 0)
    unit_step(_NCH - 1, 0)

    out_desc(_NCH - 2, 1).wait()
    out_desc(_NCH - 1, 0).wait()


_gather_call = functools.partial(
    pl.kernel,
    out_type=jax.ShapeDtypeStruct((_LEN, _D // 8, _B // 128, 8, 128),
                                  jnp.float32),
    mesh=plsc.VectorSubcoreMesh(core_axis_name="c", subcore_axis_name="s"),
    compiler_params=pltpu.CompilerParams(use_tc_tiling_on_sc=False,
                                         needs_layout_passes=False),
    scratch_types=[
        pltpu.VMEM((_LEN, _SPW), jnp.int32),
        pltpu.VMEM((2 * _NCH, 128), jnp.int32),
        pltpu.VMEM((2, 2 * 128, _D), jnp.float32),
        pltpu.VMEM((2, _CH, _D // 8, 8, _SPW + 1), jnp.float32),
        pltpu.VMEM((_D,), jnp.float32),
        pltpu.VMEM((_D,), jnp.float32),
        pltpu.SemaphoreType.DMA((2,)),
        pltpu.SemaphoreType.DMA((2,)),
    ],
)(_sc_body)


@jax.jit
def kernel(x, node_table, pre_table):
    out5 = _gather_call(x.T.astype(jnp.int32), node_table, pre_table)
    return out5.transpose(2, 4, 0, 1, 3).reshape(_B, _LEN, _D)


# speedup vs baseline: 1.9578x; 1.0061x over previous
import functools

import jax
import jax.numpy as jnp
from jax import lax
from jax.experimental import pallas as pl
from jax.experimental.pallas import tpu as pltpu
from jax.experimental.pallas import tpu_sc as plsc

_B = 1024
_LEN = 200
_D = 64
_NC, _NS = 2, 16
_NW = _NC * _NS
_SPW = _B // _NW
_CH = 8
_NCH = _LEN // _CH
_L16 = 16


def _sc_body(xt, table, pre, out, xblk, idx_v, stg_v, asm_v, cls_v, sep_v,
             gsem, osem):
    c = lax.axis_index("c")
    s = lax.axis_index("s")
    wid = s * _NC + c
    base = wid * _SPW
    tb = wid // 4
    bc0 = (wid % 4) * _SPW

    pltpu.sync_copy(xt.at[:, pl.ds(base, _SPW)], xblk)
    pltpu.sync_copy(pre.at[0], cls_v)
    pltpu.sync_copy(pre.at[1], sep_v)

    lanes = lax.iota(jnp.int32, _L16)
    hi3 = lax.shift_right_logical(lanes, 3)
    lo3 = lanes & 7

    @plsc.parallel_loop(0, _LEN, unroll=4)
    def xpose_x(p):
        r2 = 2 * (p // _CH)
        col = lanes * _CH + p % _CH
        for h in (0, 1):
            v = xblk[p, pl.ds(h * _L16, _L16)]
            plsc.store_scatter(
                idx_v, [jnp.full((_L16,), r2 + h, jnp.int32), col], v)

    def gather_descs(u, sl):
        return tuple(
            pltpu.make_async_copy(
                table.at[idx_v.at[2 * u + h]],
                stg_v.at[sl, pl.ds(h * 128, 128)],
                gsem.at[sl])
            for h in (0, 1)
        )

    def out_desc(u, sl):
        return pltpu.make_async_copy(
            asm_v.at[sl, :, :, :, pl.ds(0, _SPW)],
            out.at[pl.ds(u * _CH, _CH), :, tb, :, pl.ds(bc0, _SPW)],
            osem.at[sl])

    def xpose_unit(sl):
        @plsc.parallel_loop(0, _CH, unroll=4)
        def body(pp):
            pp_vec = jnp.full((_L16,), pp, jnp.int32)
            for h in (0, 1):
                for qg in range(_L16):
                    q_vec = jnp.full((_L16,), h * _L16 + qg, jnp.int32)
                    row = h * 128 + qg * _CH
                    for d0 in range(_D // _L16):
                        src = stg_v[sl, row + pp, pl.ds(d0 * _L16, _L16)]
                        plsc.store_scatter(
                            asm_v.at[sl],
                            [pp_vec, 2 * d0 + hi3, lo3, q_vec], src)

    def fixup(sl, pp, vec):
        pp_vec = jnp.full((_L16,), pp, jnp.int32)
        for q in range(_SPW):
            q_vec = jnp.full((_L16,), q, jnp.int32)
            for d0 in range(_D // _L16):
                plsc.store_scatter(
                    asm_v.at[sl],
                    [pp_vec, 2 * d0 + hi3, lo3, q_vec],
                    vec[pl.ds(d0 * _L16, _L16)])

    for u in (0, 1):
        for d in gather_descs(u, u % 2):
            d.start()

    def unit_step(u, sl):
        @pl.when(u >= 2)
        def _():
            out_desc(u - 2, sl).wait()

        for d in gather_descs(u, sl):
            d.wait()
        xpose_unit(sl)

        @pl.when(u == 0)
        def _():
            fixup(sl, 0, cls_v)

        @pl.when(u == _NCH - 1)
        def _():
            fixup(sl, _CH - 1, sep_v)

        out_desc(u, sl).start()

        @pl.when(u + 2 < _NCH)
        def _():
            for d in gather_descs(u + 2, sl):
                d.start()

    def pair_step(i, carry):
        for b in (0, 1):
            unit_step(2 * i + b, b)
        return carry

    lax.fori_loop(0, _NCH // 2, pair_step, 0)
    unit_step(_NCH - 1, 0)

    out_desc(_NCH - 2, 1).wait()
    out_desc(_NCH - 1, 0).wait()


_gather_call = functools.partial(
    pl.kernel,
    out_type=jax.ShapeDtypeStruct((_LEN, _D // 8, _B // 128, 8, 128),
                                  jnp.float32),
    mesh=plsc.VectorSubcoreMesh(core_axis_name="c", subcore_axis_name="s"),
    compiler_params=pltpu.CompilerParams(use_tc_tiling_on_sc=False,
                                         needs_layout_passes=False),
    scratch_types=[
        pltpu.VMEM((_LEN, _SPW), jnp.int32),
        pltpu.VMEM((2 * _NCH, 128), jnp.int32),
        pltpu.VMEM((2, 2 * 128, _D), jnp.float32),
        pltpu.VMEM((2, _CH, _D // 8, 8, _SPW + 1), jnp.float32),
        pltpu.VMEM((_D,), jnp.float32),
        pltpu.VMEM((_D,), jnp.float32),
        pltpu.SemaphoreType.DMA((2,)),
        pltpu.SemaphoreType.DMA((2,)),
    ],
)(_sc_body)


@jax.jit
def kernel(x, node_table, pre_table):
    out5 = _gather_call(x.T.astype(jnp.int32), node_table, pre_table)
    return out5.transpose(2, 4, 0, 1, 3).reshape(_B, _LEN, _D)
